# initial kernel scaffold (unmeasured)
import jax
import jax.numpy as jnp
from jax import lax
from jax.experimental import pallas as pl
from jax.experimental.pallas import tpu as pltpu

N_DEV = 8


def kernel(x, w_mat):
    m_per, k = x.shape
    _, n = w_mat.shape
    n_per = n // N_DEV

    def body(x_ref, w_ref, out_ref, w_buf, send_buf,
             copy_sems, send_sems, recv_sems):
        my = lax.axis_index("i")

        def w_tile_copy(step, slot):
            dst = lax.rem(my + step, N_DEV)
            return pltpu.make_async_copy(
                w_ref.at[:, pl.ds(dst * n_per, n_per)],
                w_buf.at[slot],
                copy_sems.at[slot],
            )

        w_tile_copy(0, 0).start()

        for s in range(N_DEV):
            slot = s % 2
            w_tile_copy(s, slot).wait()
            if s + 1 < N_DEV:
                w_tile_copy(s + 1, (s + 1) % 2).start()

            y = jnp.dot(x_ref[:, :], w_buf[slot],
                        preferred_element_type=jnp.float32)
            y = (y * jax.nn.sigmoid(y)).astype(out_ref.dtype)

            if s == 0:
                out_ref[pl.ds(my * m_per, m_per), :] = y
            else:
                send_buf[s, :, :] = y
                dst = lax.rem(my + s, N_DEV)
                pltpu.make_async_remote_copy(
                    src_ref=send_buf.at[s],
                    dst_ref=out_ref.at[pl.ds(my * m_per, m_per), :],
                    send_sem=send_sems.at[s],
                    recv_sem=recv_sems.at[s],
                    device_id=(dst,),
                    device_id_type=pl.DeviceIdType.MESH,
                ).start()

        for s in range(1, N_DEV):
            src = lax.rem(my - s + N_DEV, N_DEV)
            dst = lax.rem(my + s, N_DEV)
            rdma = pltpu.make_async_remote_copy(
                src_ref=send_buf.at[s],
                dst_ref=out_ref.at[pl.ds(src * m_per, m_per), :],
                send_sem=send_sems.at[s],
                recv_sem=recv_sems.at[s],
                device_id=(dst,),
                device_id_type=pl.DeviceIdType.MESH,
            )
            rdma.wait_send()
            rdma.wait_recv()

    return pl.pallas_call(
        body,
        out_shape=jax.ShapeDtypeStruct((N_DEV * m_per, n_per), jnp.bfloat16),
        in_specs=[
            pl.BlockSpec(memory_space=pltpu.VMEM),
            pl.BlockSpec(memory_space=pltpu.ANY),
        ],
        out_specs=pl.BlockSpec(memory_space=pltpu.VMEM),
        scratch_shapes=[
            pltpu.VMEM((2, k, n_per), jnp.bfloat16),
            pltpu.VMEM((N_DEV, m_per, n_per), jnp.bfloat16),
            pltpu.SemaphoreType.DMA((2,)),
            pltpu.SemaphoreType.DMA((N_DEV,)),
            pltpu.SemaphoreType.DMA((N_DEV,)),
        ],
        compiler_params=pltpu.CompilerParams(collective_id=0),
    )(x, w_mat)


# baseline (device time: 215959 ns/iter reference)
import jax
import jax.numpy as jnp
from jax import lax
from jax.experimental import pallas as pl
from jax.experimental.pallas import tpu as pltpu

N_DEV = 8


def kernel(x, w_mat):
    m_per, k = x.shape
    _, n = w_mat.shape
    n_per = n // N_DEV

    x = x.astype(jnp.bfloat16)
    w_mat = w_mat.astype(jnp.bfloat16)

    kh = k // 2
    n_slots = 4
    n_copies = 2 * N_DEV

    def body(x_ref, w_ref, out_ref, w_buf, send_buf,
             copy_sems, send_sems, recv_sems):
        my = lax.axis_index("i")

        def w_tile_copy(t):
            s, h = t // 2, t % 2
            dst = lax.rem(my + s, N_DEV)
            return pltpu.make_async_copy(
                w_ref.at[pl.ds(h * kh, kh), pl.ds(dst * n_per, n_per)],
                w_buf.at[t % 2],
                copy_sems.at[t % 2],
            )

        def send_rdma(s):
            dst = lax.rem(my + s, N_DEV)
            return pltpu.make_async_remote_copy(
                src_ref=send_buf.at[s % n_slots],
                dst_ref=out_ref.at[pl.ds(my * m_per, m_per), :],
                send_sem=send_sems.at[s],
                recv_sem=recv_sems.at[s],
                device_id=(dst,),
                device_id_type=pl.DeviceIdType.MESH,
            )

        w_tile_copy(0).start()

        for s in range(N_DEV):
            y = None
            for h in range(2):
                t = 2 * s + h
                w_tile_copy(t).wait()
                if t + 1 < n_copies:
                    w_tile_copy(t + 1).start()
                part = jnp.dot(x_ref[:, pl.ds(h * kh, kh)], w_buf[t % 2],
                               preferred_element_type=jnp.float32)
                y = part if y is None else y + part
            y = (y * jax.nn.sigmoid(y)).astype(out_ref.dtype)

            if s == 0:
                out_ref[pl.ds(my * m_per, m_per), :] = y
            else:
                if s >= n_slots + 1:
                    send_rdma(s - n_slots).wait_send()
                send_buf[s % n_slots, :, :] = y
                send_rdma(s).start()

        for s in range(N_DEV - n_slots, N_DEV):
            send_rdma(s).wait_send()
        for s in range(1, N_DEV):
            src = lax.rem(my - s + N_DEV, N_DEV)
            pltpu.make_async_remote_copy(
                src_ref=send_buf.at[s % n_slots],
                dst_ref=out_ref.at[pl.ds(src * m_per, m_per), :],
                send_sem=send_sems.at[s],
                recv_sem=recv_sems.at[s],
                device_id=(lax.rem(my + s, N_DEV),),
                device_id_type=pl.DeviceIdType.MESH,
            ).wait_recv()

    return pl.pallas_call(
        body,
        out_shape=jax.ShapeDtypeStruct((N_DEV * m_per, n_per), jnp.bfloat16),
        in_specs=[
            pl.BlockSpec(memory_space=pltpu.VMEM),
            pl.BlockSpec(memory_space=pl.ANY),
        ],
        out_specs=pl.BlockSpec(memory_space=pltpu.VMEM),
        scratch_shapes=[
            pltpu.VMEM((2, kh, n_per), jnp.bfloat16),
            pltpu.VMEM((n_slots, m_per, n_per), jnp.bfloat16),
            pltpu.SemaphoreType.DMA((2,)),
            pltpu.SemaphoreType.DMA((N_DEV,)),
            pltpu.SemaphoreType.DMA((N_DEV,)),
        ],
        compiler_params=pltpu.CompilerParams(
            vmem_limit_bytes=56 * 1024 * 1024,
        ),
    )(x, w_mat)


# device time: 143351 ns/iter; 1.5065x vs baseline; 1.5065x over previous
import jax
import jax.numpy as jnp
from jax import lax
from jax.experimental import pallas as pl
from jax.experimental.pallas import tpu as pltpu

N_DEV = 8


def kernel(x, w_mat):
    m_per, k = x.shape
    _, n = w_mat.shape
    n_per = n // N_DEV
    kq = k // 4
    n_wcopies = 4 * N_DEV
    xc = 64
    n_xchunks = m_per // xc
    n_slots = 4

    def body(x_hbm, w_hbm, out_ref, xb, xs, wf, wb, send_buf,
             xsems, wsems, send_sems, recv_sems):
        my = lax.axis_index("i")

        def w_copy(t):
            s, q = t // 4, t % 4
            dst = lax.rem(my + s, N_DEV)
            return pltpu.make_async_copy(
                w_hbm.at[pl.ds(q * kq, kq), pl.ds(dst * n_per, n_per)],
                wf.at[t % 2],
                wsems.at[t % 2],
            )

        def x_copy(c):
            return pltpu.make_async_copy(
                x_hbm.at[pl.ds(c * xc, xc), :],
                xs.at[c % 2],
                xsems.at[c % 2],
            )

        def send_rdma(s):
            dst = lax.rem(my + s, N_DEV)
            return pltpu.make_async_remote_copy(
                src_ref=send_buf.at[s % n_slots],
                dst_ref=out_ref.at[pl.ds(my * m_per, m_per), :],
                send_sem=send_sems.at[s],
                recv_sem=recv_sems.at[s],
                device_id=(dst,),
                device_id_type=pl.DeviceIdType.MESH,
            )

        w_copy(0).start()
        x_copy(0).start()
        x_copy(1).start()
        for c in range(n_xchunks):
            x_copy(c).wait()
            xb[pl.ds(c * xc, xc), :] = xs[c % 2].astype(xb.dtype)
            if c + 2 < n_xchunks:
                x_copy(c + 2).start()

        for s in range(N_DEV):
            y = None
            for q in range(4):
                t = 4 * s + q
                w_copy(t).wait()
                if t + 1 < n_wcopies:
                    w_copy(t + 1).start()
                wb[t % 2, :, :] = wf[t % 2].astype(wb.dtype)
                part = jnp.dot(xb[:, q * kq:(q + 1) * kq], wb[t % 2],
                               preferred_element_type=jnp.float32)
                y = part if y is None else y + part
            y = (y * jax.nn.sigmoid(y)).astype(out_ref.dtype)

            if s == 0:
                out_ref[pl.ds(my * m_per, m_per), :] = y
            else:
                if s >= n_slots + 1:
                    send_rdma(s - n_slots).wait_send()
                send_buf[s % n_slots, :, :] = y
                send_rdma(s).start()

        for s in range(N_DEV - n_slots, N_DEV):
            send_rdma(s).wait_send()
        for s in range(1, N_DEV):
            src = lax.rem(my - s + N_DEV, N_DEV)
            pltpu.make_async_remote_copy(
                src_ref=send_buf.at[s % n_slots],
                dst_ref=out_ref.at[pl.ds(src * m_per, m_per), :],
                send_sem=send_sems.at[s],
                recv_sem=recv_sems.at[s],
                device_id=(lax.rem(my + s, N_DEV),),
                device_id_type=pl.DeviceIdType.MESH,
            ).wait_recv()

    return pl.pallas_call(
        body,
        out_shape=jax.ShapeDtypeStruct((N_DEV * m_per, n_per), jnp.bfloat16),
        in_specs=[
            pl.BlockSpec(memory_space=pl.ANY),
            pl.BlockSpec(memory_space=pl.ANY),
        ],
        out_specs=pl.BlockSpec(memory_space=pltpu.VMEM),
        scratch_shapes=[
            pltpu.VMEM((m_per, k), jnp.bfloat16),
            pltpu.VMEM((2, xc, k), jnp.float32),
            pltpu.VMEM((2, kq, n_per), jnp.float32),
            pltpu.VMEM((2, kq, n_per), jnp.bfloat16),
            pltpu.VMEM((n_slots, m_per, n_per), jnp.bfloat16),
            pltpu.SemaphoreType.DMA((2,)),
            pltpu.SemaphoreType.DMA((2,)),
            pltpu.SemaphoreType.DMA((N_DEV,)),
            pltpu.SemaphoreType.DMA((N_DEV,)),
        ],
        compiler_params=pltpu.CompilerParams(
            vmem_limit_bytes=64 * 1024 * 1024,
        ),
    )(x, w_mat)


# device time: 139697 ns/iter; 1.5459x vs baseline; 1.0262x over previous
import jax
import jax.numpy as jnp
from jax import lax
from jax.experimental import pallas as pl
from jax.experimental.pallas import tpu as pltpu

N_DEV = 8


def kernel(x, w_mat):
    m_per, k = x.shape
    _, n = w_mat.shape
    n_per = n // N_DEV
    kq = k // 4
    n_wcopies = 4 * N_DEV
    xc = 512
    n_xchunks = k // xc
    n_slots = 4

    def body(x_hbm, w_hbm, out_ref, xb, xs, wf, wb, send_buf,
             xsems, wsems, send_sems, recv_sems):
        my = lax.axis_index("i")

        def w_copy(t):
            s, q = t // 4, t % 4
            dst = lax.rem(my + s, N_DEV)
            return pltpu.make_async_copy(
                w_hbm.at[pl.ds(q * kq, kq), pl.ds(dst * n_per, n_per)],
                wf.at[t % 2],
                wsems.at[t % 2],
            )

        def x_copy(c):
            return pltpu.make_async_copy(
                x_hbm.at[:, pl.ds(c * xc, xc)],
                xs.at[c % 2],
                xsems.at[c % 2],
            )

        def send_rdma(s):
            dst = lax.rem(my + s, N_DEV)
            return pltpu.make_async_remote_copy(
                src_ref=send_buf.at[s % n_slots],
                dst_ref=out_ref.at[pl.ds(my * m_per, m_per), :],
                send_sem=send_sems.at[s],
                recv_sem=recv_sems.at[s],
                device_id=(dst,),
                device_id_type=pl.DeviceIdType.MESH,
            )

        w_copy(0).start()
        x_copy(0).start()
        x_copy(1).start()

        for s in range(N_DEV):
            y = None
            for q in range(4):
                t = 4 * s + q
                if s == 0:
                    for c in range(4 * q, 4 * q + 4):
                        x_copy(c).wait()
                        xb[:, pl.ds(c * xc, xc)] = xs[c % 2].astype(xb.dtype)
                        if c + 2 < n_xchunks:
                            x_copy(c + 2).start()
                w_copy(t).wait()
                if t + 1 < n_wcopies:
                    w_copy(t + 1).start()
                wb[t % 2, :, :] = wf[t % 2].astype(wb.dtype)
                part = jnp.dot(xb[:, q * kq:(q + 1) * kq], wb[t % 2],
                               preferred_element_type=jnp.float32)
                y = part if y is None else y + part
            y = (y * jax.nn.sigmoid(y)).astype(out_ref.dtype)

            if s == 0:
                out_ref[pl.ds(my * m_per, m_per), :] = y
            else:
                if s >= n_slots + 1:
                    send_rdma(s - n_slots).wait_send()
                send_buf[s % n_slots, :, :] = y
                send_rdma(s).start()

        for s in range(N_DEV - n_slots, N_DEV):
            send_rdma(s).wait_send()
        for s in range(1, N_DEV):
            src = lax.rem(my - s + N_DEV, N_DEV)
            pltpu.make_async_remote_copy(
                src_ref=send_buf.at[s % n_slots],
                dst_ref=out_ref.at[pl.ds(src * m_per, m_per), :],
                send_sem=send_sems.at[s],
                recv_sem=recv_sems.at[s],
                device_id=(lax.rem(my + s, N_DEV),),
                device_id_type=pl.DeviceIdType.MESH,
            ).wait_recv()

    return pl.pallas_call(
        body,
        out_shape=jax.ShapeDtypeStruct((N_DEV * m_per, n_per), jnp.bfloat16),
        in_specs=[
            pl.BlockSpec(memory_space=pl.ANY),
            pl.BlockSpec(memory_space=pl.ANY),
        ],
        out_specs=pl.BlockSpec(memory_space=pltpu.VMEM),
        scratch_shapes=[
            pltpu.VMEM((m_per, k), jnp.bfloat16),
            pltpu.VMEM((2, m_per, xc), jnp.float32),
            pltpu.VMEM((2, kq, n_per), jnp.float32),
            pltpu.VMEM((2, kq, n_per), jnp.bfloat16),
            pltpu.VMEM((n_slots, m_per, n_per), jnp.bfloat16),
            pltpu.SemaphoreType.DMA((2,)),
            pltpu.SemaphoreType.DMA((2,)),
            pltpu.SemaphoreType.DMA((N_DEV,)),
            pltpu.SemaphoreType.DMA((N_DEV,)),
        ],
        compiler_params=pltpu.CompilerParams(
            vmem_limit_bytes=64 * 1024 * 1024,
        ),
    )(x, w_mat)


# device time: 129670 ns/iter; 1.6655x vs baseline; 1.0773x over previous
import jax
import jax.numpy as jnp
from jax import lax
from jax.experimental import pallas as pl
from jax.experimental.pallas import tpu as pltpu

N_DEV = 8


def kernel(x, w_mat):
    m_per, k = x.shape
    _, n = w_mat.shape
    n_per = n // N_DEV
    kq = k // 4
    n_wcopies = 4 * N_DEV
    xc = 512
    n_xchunks = k // xc
    n_slots = 4

    def body(x_hbm, w_hbm, out_ref, xb, xs, wf, send_buf,
             xsems, wsems, send_sems, recv_sems):
        my = lax.axis_index("i")

        def w_copy(t):
            s, q = t // 4, t % 4
            dst = lax.rem(my + s, N_DEV)
            return pltpu.make_async_copy(
                w_hbm.at[pl.ds(q * kq, kq), pl.ds(dst * n_per, n_per)],
                wf.at[t % 3],
                wsems.at[t % 3],
            )

        def x_copy(c):
            return pltpu.make_async_copy(
                x_hbm.at[:, pl.ds(c * xc, xc)],
                xs.at[c % 2],
                xsems.at[c % 2],
            )

        def send_rdma(s):
            dst = lax.rem(my + s, N_DEV)
            return pltpu.make_async_remote_copy(
                src_ref=send_buf.at[s % n_slots],
                dst_ref=out_ref.at[pl.ds(my * m_per, m_per), :],
                send_sem=send_sems.at[s],
                recv_sem=recv_sems.at[s],
                device_id=(dst,),
                device_id_type=pl.DeviceIdType.MESH,
            )

        w_copy(0).start()
        w_copy(1).start()
        x_copy(0).start()
        x_copy(1).start()

        for s in range(N_DEV):
            y = None
            for q in range(4):
                t = 4 * s + q
                if s == 0:
                    for c in range(4 * q, 4 * q + 4):
                        x_copy(c).wait()
                        xb[:, pl.ds(c * xc, xc)] = xs[c % 2].astype(xb.dtype)
                        if c + 2 < n_xchunks:
                            x_copy(c + 2).start()
                w_copy(t).wait()
                if t + 2 < n_wcopies:
                    w_copy(t + 2).start()
                part = jnp.dot(xb[:, q * kq:(q + 1) * kq],
                               wf[t % 3].astype(jnp.bfloat16),
                               preferred_element_type=jnp.float32)
                y = part if y is None else y + part
            y = (y * jax.nn.sigmoid(y)).astype(out_ref.dtype)

            if s == 0:
                out_ref[pl.ds(my * m_per, m_per), :] = y
            else:
                if s >= n_slots + 1:
                    send_rdma(s - n_slots).wait_send()
                send_buf[s % n_slots, :, :] = y
                send_rdma(s).start()

        for s in range(N_DEV - n_slots, N_DEV):
            send_rdma(s).wait_send()
        for s in range(1, N_DEV):
            src = lax.rem(my - s + N_DEV, N_DEV)
            pltpu.make_async_remote_copy(
                src_ref=send_buf.at[s % n_slots],
                dst_ref=out_ref.at[pl.ds(src * m_per, m_per), :],
                send_sem=send_sems.at[s],
                recv_sem=recv_sems.at[s],
                device_id=(lax.rem(my + s, N_DEV),),
                device_id_type=pl.DeviceIdType.MESH,
            ).wait_recv()

    return pl.pallas_call(
        body,
        out_shape=jax.ShapeDtypeStruct((N_DEV * m_per, n_per), jnp.bfloat16),
        in_specs=[
            pl.BlockSpec(memory_space=pl.ANY),
            pl.BlockSpec(memory_space=pl.ANY),
        ],
        out_specs=pl.BlockSpec(memory_space=pltpu.VMEM),
        scratch_shapes=[
            pltpu.VMEM((m_per, k), jnp.bfloat16),
            pltpu.VMEM((2, m_per, xc), jnp.float32),
            pltpu.VMEM((3, kq, n_per), jnp.float32),
            pltpu.VMEM((n_slots, m_per, n_per), jnp.bfloat16),
            pltpu.SemaphoreType.DMA((2,)),
            pltpu.SemaphoreType.DMA((3,)),
            pltpu.SemaphoreType.DMA((N_DEV,)),
            pltpu.SemaphoreType.DMA((N_DEV,)),
        ],
        compiler_params=pltpu.CompilerParams(
            vmem_limit_bytes=64 * 1024 * 1024,
        ),
    )(x, w_mat)


# device time: 119577 ns/iter; 1.8060x vs baseline; 1.0844x over previous
import jax
import jax.numpy as jnp
from jax import lax
from jax.experimental import pallas as pl
from jax.experimental.pallas import tpu as pltpu

N_DEV = 8


def kernel(x, w_mat):
    m_per, k = x.shape
    _, n = w_mat.shape
    n_per = n // N_DEV
    kq = k // 4
    n_wcopies = 4 * N_DEV
    xc = 512
    n_xchunks = k // xc
    n_slots = 4

    def body(x_hbm, w_hbm, out_ref, xb, xs, wf, send_buf,
             xsems, wsems, send_sems, recv_sems):
        my = lax.axis_index("i")

        def w_copy(t):
            s, q = t // 4, t % 4
            dst = lax.rem(my + s + 1, N_DEV)
            return pltpu.make_async_copy(
                w_hbm.at[pl.ds(q * kq, kq), pl.ds(dst * n_per, n_per)],
                wf.at[t % 3],
                wsems.at[t % 3],
            )

        def x_copy(c):
            return pltpu.make_async_copy(
                x_hbm.at[:, pl.ds(c * xc, xc)],
                xs.at[c % 2],
                xsems.at[c % 2],
            )

        def send_rdma(s):
            dst = lax.rem(my + s + 1, N_DEV)
            return pltpu.make_async_remote_copy(
                src_ref=send_buf.at[s % n_slots],
                dst_ref=out_ref.at[pl.ds(my * m_per, m_per), :],
                send_sem=send_sems.at[s],
                recv_sem=recv_sems.at[s],
                device_id=(dst,),
                device_id_type=pl.DeviceIdType.MESH,
            )

        w_copy(0).start()
        w_copy(1).start()
        x_copy(0).start()
        x_copy(1).start()

        for s in range(N_DEV):
            y = None
            for q in range(4):
                t = 4 * s + q
                if s == 0:
                    for c in range(4 * q, 4 * q + 4):
                        x_copy(c).wait()
                        xb[:, pl.ds(c * xc, xc)] = xs[c % 2].astype(xb.dtype)
                        if c + 2 < n_xchunks:
                            x_copy(c + 2).start()
                w_copy(t).wait()
                if t + 2 < n_wcopies:
                    w_copy(t + 2).start()
                part = jnp.dot(xb[:, q * kq:(q + 1) * kq],
                               wf[t % 3].astype(jnp.bfloat16),
                               preferred_element_type=jnp.float32)
                y = part if y is None else y + part
            y = (y * jax.nn.sigmoid(y)).astype(out_ref.dtype)

            if s == N_DEV - 1:
                out_ref[pl.ds(my * m_per, m_per), :] = y
            else:
                if s >= n_slots:
                    send_rdma(s - n_slots).wait_send()
                send_buf[s % n_slots, :, :] = y
                send_rdma(s).start()

        for s in range(N_DEV - 1 - n_slots, N_DEV - 1):
            send_rdma(s).wait_send()
        for s in range(N_DEV - 1):
            src = lax.rem(my - s - 1 + N_DEV, N_DEV)
            pltpu.make_async_remote_copy(
                src_ref=send_buf.at[s % n_slots],
                dst_ref=out_ref.at[pl.ds(src * m_per, m_per), :],
                send_sem=send_sems.at[s],
                recv_sem=recv_sems.at[s],
                device_id=(lax.rem(my + s + 1, N_DEV),),
                device_id_type=pl.DeviceIdType.MESH,
            ).wait_recv()

    return pl.pallas_call(
        body,
        out_shape=jax.ShapeDtypeStruct((N_DEV * m_per, n_per), jnp.bfloat16),
        in_specs=[
            pl.BlockSpec(memory_space=pl.ANY),
            pl.BlockSpec(memory_space=pl.ANY),
        ],
        out_specs=pl.BlockSpec(memory_space=pltpu.VMEM),
        scratch_shapes=[
            pltpu.VMEM((m_per, k), jnp.bfloat16),
            pltpu.VMEM((2, m_per, xc), jnp.float32),
            pltpu.VMEM((3, kq, n_per), jnp.float32),
            pltpu.VMEM((n_slots, m_per, n_per), jnp.bfloat16),
            pltpu.SemaphoreType.DMA((2,)),
            pltpu.SemaphoreType.DMA((3,)),
            pltpu.SemaphoreType.DMA((N_DEV,)),
            pltpu.SemaphoreType.DMA((N_DEV,)),
        ],
        compiler_params=pltpu.CompilerParams(
            vmem_limit_bytes=64 * 1024 * 1024,
        ),
    )(x, w_mat)


# device time: 118201 ns/iter; 1.8270x vs baseline; 1.0116x over previous
import jax
import jax.numpy as jnp
from jax import lax
from jax.experimental import pallas as pl
from jax.experimental.pallas import tpu as pltpu

N_DEV = 8


def kernel(x, w_mat):
    m_per, k = x.shape
    _, n = w_mat.shape
    n_per = n // N_DEV
    kq = k // 4
    n_wcopies = 4 * N_DEV
    xc = 512
    n_xchunks = k // xc
    n_slots = 4

    def body(x_hbm, w_hbm, out_ref, xb, xs, wf, send_buf,
             xsems, wsems, send_sems, recv_sems):
        my = lax.axis_index("i")

        def w_copy(t):
            s, q = t // 4, t % 4
            dst = lax.rem(my + s + 1, N_DEV)
            return pltpu.make_async_copy(
                w_hbm.at[pl.ds(q * kq, kq), pl.ds(dst * n_per, n_per)],
                wf.at[t % 4],
                wsems.at[t % 4],
            )

        def x_copy(c):
            return pltpu.make_async_copy(
                x_hbm.at[:, pl.ds(c * xc, xc)],
                xs.at[c % 2],
                xsems.at[c % 2],
            )

        def send_rdma(s):
            dst = lax.rem(my + s + 1, N_DEV)
            return pltpu.make_async_remote_copy(
                src_ref=send_buf.at[s % n_slots],
                dst_ref=out_ref.at[pl.ds(my * m_per, m_per), :],
                send_sem=send_sems.at[s],
                recv_sem=recv_sems.at[s],
                device_id=(dst,),
                device_id_type=pl.DeviceIdType.MESH,
            )

        w_copy(0).start()
        w_copy(1).start()
        w_copy(2).start()
        x_copy(0).start()
        x_copy(1).start()

        for s in range(N_DEV):
            y = None
            for q in range(4):
                t = 4 * s + q
                if s == 0:
                    for c in range(4 * q, 4 * q + 4):
                        x_copy(c).wait()
                        xb[:, pl.ds(c * xc, xc)] = xs[c % 2].astype(xb.dtype)
                        if c + 2 < n_xchunks:
                            x_copy(c + 2).start()
                w_copy(t).wait()
                if t + 3 < n_wcopies:
                    w_copy(t + 3).start()
                part = jnp.dot(xb[:, q * kq:(q + 1) * kq],
                               wf[t % 4].astype(jnp.bfloat16),
                               preferred_element_type=jnp.float32)
                y = part if y is None else y + part
            y = (y * jax.nn.sigmoid(y)).astype(out_ref.dtype)

            if s == N_DEV - 1:
                send_rdma(3).wait_send()
                send_buf[3, :, :] = y
                pltpu.make_async_copy(
                    send_buf.at[3],
                    out_ref.at[pl.ds(my * m_per, m_per), :],
                    xsems.at[0],
                ).start()
            else:
                if s >= n_slots:
                    send_rdma(s - n_slots).wait_send()
                send_buf[s % n_slots, :, :] = y
                send_rdma(s).start()

        for s in range(N_DEV - n_slots, N_DEV - 1):
            send_rdma(s).wait_send()
        pltpu.make_async_copy(
            send_buf.at[3],
            out_ref.at[pl.ds(my * m_per, m_per), :],
            xsems.at[0],
        ).wait()
        for s in range(N_DEV - 1):
            src = lax.rem(my - s - 1 + N_DEV, N_DEV)
            pltpu.make_async_remote_copy(
                src_ref=send_buf.at[s % n_slots],
                dst_ref=out_ref.at[pl.ds(src * m_per, m_per), :],
                send_sem=send_sems.at[s],
                recv_sem=recv_sems.at[s],
                device_id=(lax.rem(my + s + 1, N_DEV),),
                device_id_type=pl.DeviceIdType.MESH,
            ).wait_recv()

    return pl.pallas_call(
        body,
        out_shape=jax.ShapeDtypeStruct((N_DEV * m_per, n_per), jnp.bfloat16),
        in_specs=[
            pl.BlockSpec(memory_space=pl.ANY),
            pl.BlockSpec(memory_space=pl.ANY),
        ],
        out_specs=pl.BlockSpec(memory_space=pl.ANY),
        scratch_shapes=[
            pltpu.VMEM((m_per, k), jnp.bfloat16),
            pltpu.VMEM((2, m_per, xc), jnp.float32),
            pltpu.VMEM((4, kq, n_per), jnp.float32),
            pltpu.VMEM((n_slots, m_per, n_per), jnp.bfloat16),
            pltpu.SemaphoreType.DMA((2,)),
            pltpu.SemaphoreType.DMA((4,)),
            pltpu.SemaphoreType.DMA((N_DEV,)),
            pltpu.SemaphoreType.DMA((N_DEV,)),
        ],
        compiler_params=pltpu.CompilerParams(
            vmem_limit_bytes=64 * 1024 * 1024,
        ),
    )(x, w_mat)
